# baseline (speedup 1.0000x reference)
import functools

import jax
import jax.numpy as jnp
from jax import lax
from jax.experimental import pallas as pl
from jax.experimental.pallas import tpu as pltpu
from jax.experimental.pallas import tpu_sc as plsc

ALPHA = 0.2
NC = 2
NS = 16
NW = NC * NS
LANES = 16
NB = 3


def _sc_scatter_partials(N, D, E, K):
    assert E % NW == 0
    epw = E // NW
    assert epw % K == 0 and K % LANES == 0
    n_chunks = epw // K
    assert n_chunks >= 6
    rows_per_tile = (N // NS) // 8 * 8
    last_rows = N - rows_per_tile * (NS - 1)

    mesh = plsc.VectorSubcoreMesh(core_axis_name="c", subcore_axis_name="s")

    @functools.partial(
        pl.kernel,
        mesh=mesh,
        out_type=[
            jax.ShapeDtypeStruct((N, D), jnp.float32),
            jax.ShapeDtypeStruct((N, D), jnp.float32),
        ],
        scratch_types=[
            pltpu.VMEM((n_chunks, K), jnp.float32),
            pltpu.VMEM((NB, K, D), jnp.float32),
            pltpu.VMEM((K,), jnp.int32),
            pltpu.VMEM((K,), jnp.int32),
            pltpu.VMEM((K,), jnp.int32),
            pltpu.VMEM((K,), jnp.int32),
            pltpu.VMEM((K,), jnp.int32),
            pltpu.VMEM((K,), jnp.int32),
            pltpu.VMEM_SHARED((N, D), jnp.float32),
            pltpu.SemaphoreType.DMA,
            pltpu.SemaphoreType.DMA,
            pltpu.SemaphoreType.DMA,
            pltpu.SemaphoreType.DMA,
            pltpu.SemaphoreType.DMA,
            pltpu.SemaphoreType.DMA,
            pltpu.SemaphoreType.DMA,
            pltpu.SemaphoreType.DMA,
            pltpu.SemaphoreType.DMA,
            pltpu.SemaphoreType.DMA,
        ],
        compiler_params=pltpu.CompilerParams(needs_layout_passes=False),
    )
    def kern(src_hbm, dst_hbm, w_hbm, h_hbm, zeros_hbm, p0_hbm, p1_hbm,
             w_all, rows, sv0, sv1, sv2, dv0, dv1, dv2, agg_sh,
             si, sf0, sf1, sf2, sg0, sg1, sg2, ss0, ss1, ss2):
        sv = (sv0, sv1, sv2)
        dv = (dv0, dv1, dv2)
        sf = (sf0, sf1, sf2)
        sg = (sg0, sg1, sg2)
        ss = (ss0, ss1, ss2)
        cid = lax.axis_index("c")
        sid = lax.axis_index("s")
        wid = cid * NS + sid

        def copy_tile_rows(src_ref, dst_ref):
            @pl.when(sid < NS - 1)
            def _():
                base = sid * rows_per_tile
                pltpu.sync_copy(src_ref.at[pl.ds(base, rows_per_tile)],
                                dst_ref.at[pl.ds(base, rows_per_tile)])

            @pl.when(sid == NS - 1)
            def _():
                base = (NS - 1) * rows_per_tile
                pltpu.sync_copy(src_ref.at[pl.ds(base, last_rows)],
                                dst_ref.at[pl.ds(base, last_rows)])

        dw = pltpu.async_copy(w_hbm.at[wid], w_all, si)
        copy_tile_rows(zeros_hbm, agg_sh)
        dw.wait()
        plsc.subcore_barrier()

        e0 = wid * epw

        def start_fetch(j, b):
            base = e0 + j * K
            pltpu.async_copy(src_hbm.at[pl.ds(base, K)], sv[b], sf[b])
            pltpu.async_copy(dst_hbm.at[pl.ds(base, K)], dv[b], sf[b])

        def wait_fetch(b):
            pltpu.make_async_copy(src_hbm.at[pl.ds(0, K)], sv[b], sf[b]).wait()
            pltpu.make_async_copy(dst_hbm.at[pl.ds(0, K)], dv[b], sf[b]).wait()

        def start_gather(b):
            pltpu.async_copy(h_hbm.at[sv[b]], rows.at[b], sg[b])

        def wait_gather(b):
            pltpu.make_async_copy(h_hbm.at[sv[b]], rows.at[b], sg[b]).wait()

        def start_scatter(b):
            pltpu.async_copy(rows.at[b], agg_sh.at[dv[b]], ss[b], add=True)

        def wait_scatter(b):
            pltpu.make_async_copy(rows.at[b], agg_sh.at[dv[b]], ss[b]).wait()

        def scale(i, b):
            rows_b = rows.at[b]

            @plsc.parallel_loop(0, K // LANES)
            def _(g):
                w16 = w_all[i, pl.ds(g * LANES, LANES)]
                for jj in range(LANES):
                    j = g * LANES + jj
                    wscal = w16[jj]
                    for c in range(D // LANES):
                        sl = pl.ds(c * LANES, LANES)
                        rows_b[j, sl] = rows_b[j, sl] * wscal

        start_fetch(0, 0)
        start_fetch(1, 1)
        wait_fetch(0)
        start_gather(0)

        def step(i, b):
            wait_gather(b)
            scale(i, b)
            b2 = (b + 2) % 3

            @pl.when(i + 2 < n_chunks)
            def _():
                @pl.when(i >= 1)
                def _():
                    wait_scatter(b2)

                start_fetch(i + 2, b2)

            b1 = (b + 1) % 3

            @pl.when(i + 1 < n_chunks)
            def _():
                wait_fetch(b1)
                start_gather(b1)

            start_scatter(b)

        def tri(t, carry):
            for u in range(3):
                i = 3 * t + u

                @pl.when(i < n_chunks)
                def _(i=i, u=u):
                    step(i, u)
            return carry

        lax.fori_loop(0, (n_chunks + 2) // 3, tri, 0)

        for i in (n_chunks - 3, n_chunks - 2, n_chunks - 1):
            wait_scatter(i % 3)
        plsc.subcore_barrier()

        @pl.when(cid == 0)
        def _():
            copy_tile_rows(agg_sh, p0_hbm)

        @pl.when(cid == 1)
        def _():
            copy_tile_rows(agg_sh, p1_hbm)

    return kern


def _blend_body(p0_ref, p1_ref, h0_ref, o_ref):
    o_ref[...] = ((1.0 - ALPHA) * (p0_ref[...] + p1_ref[...])
                  + ALPHA * h0_ref[...])


def _tc_blend(p0, p1, h0):
    N, D = h0.shape
    block = 1000
    grid = (N // block,)
    spec = pl.BlockSpec((block, D), lambda i: (i, 0))
    return pl.pallas_call(
        _blend_body,
        grid=grid,
        in_specs=[spec, spec, spec],
        out_specs=spec,
        out_shape=jax.ShapeDtypeStruct((N, D), jnp.float32),
    )(p0, p1, h0)


@jax.jit
def kernel(edge_index, edge_weight, h, h0):
    N, D = h.shape
    E = edge_weight.shape[0]
    K = 80
    n_chunks = E // (NW * K)
    src = edge_index[0].astype(jnp.int32)
    dst = edge_index[1].astype(jnp.int32)
    w = edge_weight.reshape(NW, n_chunks, K)
    zeros = jnp.zeros((N, D), jnp.float32)
    p0, p1 = _sc_scatter_partials(N, D, E, K)(src, dst, w, h, zeros)
    return _tc_blend(p0, p1, h0[:N])


# speedup vs baseline: 1.3481x; 1.3481x over previous
import functools

import jax
import jax.numpy as jnp
from jax import lax
from jax.experimental import pallas as pl
from jax.experimental.pallas import tpu as pltpu
from jax.experimental.pallas import tpu_sc as plsc

ALPHA = 0.2
NC = 2
NS = 16
NW = NC * NS
LANES = 16
NB = 3
GD = NB - 2
FD = NB - 1


def _sc_scatter_partials(N, D, E, K):
    assert E % NW == 0
    epw = E // NW
    assert epw % K == 0 and K % LANES == 0
    n_chunks = epw // K
    assert n_chunks >= 2 * NB
    rows_per_tile = (N // NS) // 8 * 8
    last_rows = N - rows_per_tile * (NS - 1)

    mesh = plsc.VectorSubcoreMesh(core_axis_name="c", subcore_axis_name="s")

    scratch_types = (
        [pltpu.VMEM((n_chunks, K), jnp.float32),
         pltpu.VMEM((NB, K, D), jnp.float32)]
        + [pltpu.VMEM((K,), jnp.int32)] * NB
        + [pltpu.VMEM((K,), jnp.int32)] * NB
        + [pltpu.VMEM_SHARED((N, D), jnp.float32)]
        + [pltpu.SemaphoreType.DMA] * (1 + 3 * NB)
    )

    @functools.partial(
        pl.kernel,
        mesh=mesh,
        out_type=[
            jax.ShapeDtypeStruct((N, D), jnp.float32),
            jax.ShapeDtypeStruct((N, D), jnp.float32),
        ],
        scratch_types=scratch_types,
        compiler_params=pltpu.CompilerParams(needs_layout_passes=False),
    )
    def kern(src_hbm, dst_hbm, w_hbm, h_hbm, zeros_hbm, p0_hbm, p1_hbm,
             w_all, rows, *scr):
        sv = scr[:NB]
        dv = scr[NB:2 * NB]
        agg_sh = scr[2 * NB]
        si = scr[2 * NB + 1]
        sf = scr[2 * NB + 2:2 * NB + 2 + NB]
        sg = scr[2 * NB + 2 + NB:2 * NB + 2 + 2 * NB]
        ss = scr[2 * NB + 2 + 2 * NB:]
        cid = lax.axis_index("c")
        sid = lax.axis_index("s")
        wid = cid * NS + sid

        def copy_tile_rows(src_ref, dst_ref):
            @pl.when(sid < NS - 1)
            def _():
                base = sid * rows_per_tile
                pltpu.sync_copy(src_ref.at[pl.ds(base, rows_per_tile)],
                                dst_ref.at[pl.ds(base, rows_per_tile)])

            @pl.when(sid == NS - 1)
            def _():
                base = (NS - 1) * rows_per_tile
                pltpu.sync_copy(src_ref.at[pl.ds(base, last_rows)],
                                dst_ref.at[pl.ds(base, last_rows)])

        dw = pltpu.async_copy(w_hbm.at[wid], w_all, si)
        copy_tile_rows(zeros_hbm, agg_sh)
        dw.wait()
        plsc.subcore_barrier()

        e0 = wid * epw

        def start_fetch(j, b):
            base = e0 + j * K
            pltpu.async_copy(src_hbm.at[pl.ds(base, K)], sv[b], sf[b])
            pltpu.async_copy(dst_hbm.at[pl.ds(base, K)], dv[b], sf[b])

        def wait_fetch(b):
            pltpu.make_async_copy(src_hbm.at[pl.ds(0, K)], sv[b], sf[b]).wait()
            pltpu.make_async_copy(dst_hbm.at[pl.ds(0, K)], dv[b], sf[b]).wait()

        def start_gather(b):
            pltpu.async_copy(h_hbm.at[sv[b]], rows.at[b], sg[b])

        def wait_gather(b):
            pltpu.make_async_copy(h_hbm.at[sv[b]], rows.at[b], sg[b]).wait()

        def start_scatter(b):
            pltpu.async_copy(rows.at[b], agg_sh.at[dv[b]], ss[b], add=True)

        def wait_scatter(b):
            pltpu.make_async_copy(rows.at[b], agg_sh.at[dv[b]], ss[b]).wait()

        def scale(i, b):
            rows_b = rows.at[b]

            def group(g, carry):
                w16 = w_all[i, pl.ds(g * LANES, LANES)]
                for jj in range(LANES):
                    j = g * LANES + jj
                    wscal = w16[jj]
                    for c in range(D // LANES):
                        sl = pl.ds(c * LANES, LANES)
                        rows_b[j, sl] = rows_b[j, sl] * wscal
                return carry

            lax.fori_loop(0, K // LANES, group, 0)

        for j in range(FD):
            start_fetch(j, j)
        for j in range(GD):
            wait_fetch(j)
            start_gather(j)

        def step(i, b):
            wait_gather(b)
            b1 = (b + GD) % NB

            @pl.when(i + GD < n_chunks)
            def _():
                wait_fetch(b1)
                start_gather(b1)

            scale(i, b)
            b2 = (b + FD) % NB

            @pl.when(i + FD < n_chunks)
            def _():
                @pl.when(i + FD >= NB)
                def _():
                    wait_scatter(b2)

                start_fetch(i + FD, b2)

            start_scatter(b)

        def ring(t, carry):
            for u in range(NB):
                i = NB * t + u

                @pl.when(i < n_chunks)
                def _(i=i, u=u):
                    step(i, u)
            return carry

        lax.fori_loop(0, (n_chunks + NB - 1) // NB, ring, 0)

        for i in range(n_chunks - NB, n_chunks):
            wait_scatter(i % NB)
        plsc.subcore_barrier()

        @pl.when(cid == 0)
        def _():
            copy_tile_rows(agg_sh, p0_hbm)

        @pl.when(cid == 1)
        def _():
            copy_tile_rows(agg_sh, p1_hbm)

    return kern


def _blend_body(p0_ref, p1_ref, h0_ref, o_ref):
    o_ref[...] = ((1.0 - ALPHA) * (p0_ref[...] + p1_ref[...])
                  + ALPHA * h0_ref[...])


def _tc_blend(p0, p1, h0):
    N, D = h0.shape
    block = 1000
    grid = (N // block,)
    spec = pl.BlockSpec((block, D), lambda i: (i, 0))
    return pl.pallas_call(
        _blend_body,
        grid=grid,
        in_specs=[spec, spec, spec],
        out_specs=spec,
        out_shape=jax.ShapeDtypeStruct((N, D), jnp.float32),
    )(p0, p1, h0)


@jax.jit
def kernel(edge_index, edge_weight, h, h0):
    N, D = h.shape
    E = edge_weight.shape[0]
    K = 80
    n_chunks = E // (NW * K)
    src = edge_index[0].astype(jnp.int32)
    dst = edge_index[1].astype(jnp.int32)
    w = edge_weight.reshape(NW, n_chunks, K)
    zeros = jnp.zeros((N, D), jnp.float32)
    p0, p1 = _sc_scatter_partials(N, D, E, K)(src, dst, w, h, zeros)
    return _tc_blend(p0, p1, h0[:N])
